# Initial kernel scaffold; baseline (speedup 1.0000x reference)
import jax
import jax.numpy as jnp
from jax.experimental import pallas as pl


def kernel(x, edge_index, Ws, Wc, bc):
    raise NotImplementedError("write your pallas kernel here")


# speedup vs baseline: 7.5694x; 7.5694x over previous
import functools

import jax
import jax.numpy as jnp
from jax import lax
from jax.experimental import pallas as pl
from jax.experimental.pallas import tpu as pltpu
from jax.experimental.pallas import tpu_sc as plsc

ALPHA = 0.2
THETA = 0.5

NC = 2
NS = 16
NW = NC * NS
WIN = 128


def _make_sc_agg(rows_spm: int, wpt: int, feat: int):
  per_tile = rows_spm // NS
  mesh = plsc.VectorSubcoreMesh(core_axis_name="c", subcore_axis_name="s")

  @functools.partial(
      pl.kernel,
      out_type=jax.ShapeDtypeStruct((NC, rows_spm, feat), jnp.float32),
      mesh=mesh,
      scratch_types=[
          pltpu.VMEM((wpt, WIN), jnp.int32),
          pltpu.VMEM((wpt, WIN), jnp.int32),
          pltpu.VMEM((WIN, feat), jnp.float32),
          pltpu.VMEM_SHARED((rows_spm, feat), jnp.float32),
      ],
  )
  def sc_agg(g_hbm, src_hbm, dst_hbm, zeros_hbm, out_hbm,
             src_v, dst_v, rows_v, acc_spm):
    cid = lax.axis_index("c")
    sid = lax.axis_index("s")
    pltpu.sync_copy(src_hbm.at[cid, sid], src_v)
    pltpu.sync_copy(dst_hbm.at[cid, sid], dst_v)
    pltpu.sync_copy(zeros_hbm, acc_spm.at[pl.ds(sid * per_tile, per_tile)])
    plsc.subcore_barrier()

    @pl.loop(0, wpt)
    def _(w):
      pltpu.sync_copy(g_hbm.at[src_v.at[w]], rows_v)
      pltpu.sync_copy(rows_v, acc_spm.at[dst_v.at[w]], add=True)

    plsc.subcore_barrier()
    pltpu.sync_copy(acc_spm.at[pl.ds(sid * per_tile, per_tile)],
                    out_hbm.at[cid, pl.ds(sid * per_tile, per_tile)])

  return sc_agg


def _prep_tc(deg2_ref, x_ref, dinv_ref, g0_ref):
  deg = deg2_ref[0, :, 0:1] + deg2_ref[1, :, 0:1] + 1.0
  dinv = lax.rsqrt(deg)
  dinv_ref[...] = dinv
  g0_ref[...] = dinv * x_ref[...]


def _layer_tc(s2_ref, g_ref, x_ref, dinv_ref, w_ref, h_ref, gout_ref):
  dinv = dinv_ref[...]
  s = s2_ref[0] + s2_ref[1]
  agg = dinv * (s + g_ref[...])
  sup = (1.0 - ALPHA) * agg + ALPHA * x_ref[...]
  h = jnp.maximum(jnp.dot(sup, w_ref[...], preferred_element_type=jnp.float32),
                  0.0)
  h_ref[...] = h
  gout_ref[...] = dinv * h


def _logits_tc(h_ref, wc_ref, bc_ref, out_ref):
  out_ref[...] = jnp.dot(h_ref[...], wc_ref[...],
                         preferred_element_type=jnp.float32) + bc_ref[...]


def kernel(x, edge_index, Ws, Wc, bc):
  n, d = x.shape
  num_layers = Ws.shape[0]
  e = edge_index.shape[1]

  wpt = -(-e // (NW * WIN))
  e_pad = NW * wpt * WIN
  rows_spm = -(-(n + 1) // (NS * 8)) * (NS * 8)
  src = edge_index[0]
  dst = edge_index[1]
  pad = e_pad - e
  src_p = jnp.concatenate([src, jnp.zeros((pad,), jnp.int32)])
  dst_p = jnp.concatenate([dst, jnp.full((pad,), n, jnp.int32)])
  src_r = src_p.reshape(NC, NS, wpt, WIN)
  dst_r = dst_p.reshape(NC, NS, wpt, WIN)

  betas = jnp.log(THETA / jnp.arange(1, num_layers + 1, dtype=x.dtype) + 1.0)
  eye = jnp.eye(d, dtype=x.dtype)
  w_eff = (1.0 - betas)[:, None, None] * eye[None] + betas[:, None, None] * Ws

  per_tile = rows_spm // NS
  zeros128 = jnp.zeros((per_tile, d), jnp.float32)
  ones128 = jnp.ones((n, d), jnp.float32)

  sc_agg_feat = _make_sc_agg(rows_spm, wpt, d)

  blk = 1000
  grid = (n // blk,)

  deg2 = sc_agg_feat(ones128, src_r, dst_r, zeros128)

  dinv, g0 = pl.pallas_call(
      _prep_tc,
      grid=grid,
      in_specs=[
          pl.BlockSpec((NC, blk, d), lambda i: (0, i, 0)),
          pl.BlockSpec((blk, d), lambda i: (i, 0)),
      ],
      out_specs=[
          pl.BlockSpec((blk, 1), lambda i: (i, 0)),
          pl.BlockSpec((blk, d), lambda i: (i, 0)),
      ],
      out_shape=[
          jax.ShapeDtypeStruct((n, 1), jnp.float32),
          jax.ShapeDtypeStruct((n, d), jnp.float32),
      ],
  )(deg2, x)

  layer_call = pl.pallas_call(
      _layer_tc,
      grid=grid,
      in_specs=[
          pl.BlockSpec((NC, blk, d), lambda i: (0, i, 0)),
          pl.BlockSpec((blk, d), lambda i: (i, 0)),
          pl.BlockSpec((blk, d), lambda i: (i, 0)),
          pl.BlockSpec((blk, 1), lambda i: (i, 0)),
          pl.BlockSpec((d, d), lambda i: (0, 0)),
      ],
      out_specs=[
          pl.BlockSpec((blk, d), lambda i: (i, 0)),
          pl.BlockSpec((blk, d), lambda i: (i, 0)),
      ],
      out_shape=[
          jax.ShapeDtypeStruct((n, d), jnp.float32),
          jax.ShapeDtypeStruct((n, d), jnp.float32),
      ],
  )

  def layer(carry, w_l):
    _h, g = carry
    s2 = sc_agg_feat(g, src_r, dst_r, zeros128)
    h_new, g_new = layer_call(s2, g, x, dinv, w_l)
    return (h_new, g_new), None

  (h_fin, _), _ = lax.scan(layer, (x, g0), w_eff)

  logits = pl.pallas_call(
      _logits_tc,
      grid=grid,
      in_specs=[
          pl.BlockSpec((blk, d), lambda i: (i, 0)),
          pl.BlockSpec((d, Wc.shape[1]), lambda i: (0, 0)),
          pl.BlockSpec((1, Wc.shape[1]), lambda i: (0, 0)),
      ],
      out_specs=pl.BlockSpec((blk, Wc.shape[1]), lambda i: (i, 0)),
      out_shape=jax.ShapeDtypeStruct((n, Wc.shape[1]), jnp.float32),
  )(h_fin, Wc, bc.reshape(1, -1))

  return logits
